# baseline (speedup 1.0000x reference)
import functools

import numpy as np
import jax
import jax.numpy as jnp
from jax import lax
from jax.experimental import pallas as pl
from jax.experimental.pallas import tpu as pltpu
from jax.experimental.pallas import tpu_sc as plsc

_BS = 512
_WT = 5
_NEG_INF = float("-inf")
_TINY = np.float32(np.finfo(np.float32).tiny)
_SPAN = np.float32(np.float32(1.0) - _TINY)
_ONE_F32_BITS = np.uint32(np.array(1.0, np.float32).view(np.uint32))


def _sc_gather_rows(table, idx):
    _, d = table.shape
    b = idx.shape[0]
    esize = np.dtype(table.dtype).itemsize
    info = plsc.get_sparse_core_info()
    nw = info.num_cores * info.num_subcores
    b_per_w = b // nw
    ch = b_per_w
    while ch * d * esize > 256 * 1024:
        ch //= 2
    n_ch = b_per_w // ch
    mesh = plsc.VectorSubcoreMesh(core_axis_name="c", subcore_axis_name="s")

    @functools.partial(
        pl.kernel,
        mesh=mesh,
        out_type=jax.ShapeDtypeStruct((b, d), table.dtype),
        scratch_types=[
            pltpu.VMEM((ch,), jnp.int32),
            pltpu.VMEM((ch, d), table.dtype),
            pltpu.SemaphoreType.DMA,
        ],
    )
    def gather_k(table_hbm, idx_hbm, out_hbm, idx_v, rows_v, sem):
        wid = lax.axis_index("s") * info.num_cores + lax.axis_index("c")
        base = wid * b_per_w
        for c in range(n_ch):
            off = base + c * ch
            pltpu.sync_copy(idx_hbm.at[pl.ds(off, ch)], idx_v)
            pltpu.async_copy(table_hbm.at[idx_v], rows_v, sem).wait()
            pltpu.sync_copy(rows_v, out_hbm.at[pl.ds(off, ch)])

    return gather_k(table, idx)


def _u8_to_i32(x):
    b, n = x.shape
    return lax.bitcast_convert_type(x.reshape(b, n // 4, 4), jnp.int32)


def _i32_to_u8(x):
    b, m = x.shape
    return lax.bitcast_convert_type(x, jnp.uint8).reshape(b, 4 * m)


def _pack_u8(adj, blk=512):
    v, n = adj.shape

    def body(x_ref, o_ref):
        o_ref[...] = (x_ref[...] != 0.0).astype(jnp.uint8)

    return pl.pallas_call(
        body,
        grid=(v // blk,),
        in_specs=[pl.BlockSpec((blk, n), lambda i: (i, 0))],
        out_specs=pl.BlockSpec((blk, n), lambda i: (i, 0)),
        out_shape=jax.ShapeDtypeStruct((v, n), jnp.uint8),
    )(adj)


def _inkernel_gumbel(keys_ref, pid, blk, n):
    i = (pid * blk) // _BS
    k1 = keys_ref[i, 0]
    k2 = keys_ref[i, 1]
    l0 = pid * blk - i * _BS
    row = l0 + lax.broadcasted_iota(jnp.int32, (blk, n), 0)
    col = lax.broadcasted_iota(jnp.int32, (blk, n), 1)
    x1 = (row * n + col).astype(jnp.uint32)
    x0 = jnp.zeros((blk, n), jnp.uint32)
    ks0, ks1 = k1, k2
    ks2 = k1 ^ k2 ^ np.uint32(0x1BD11BDA)

    def rnd(a, b, r):
        a = a + b
        b = lax.shift_left(b, np.uint32(r)) | lax.shift_right_logical(
            b, np.uint32(32 - r))
        b = a ^ b
        return a, b

    r0 = (13, 15, 26, 6)
    r1 = (17, 29, 16, 24)
    x0 = x0 + ks0
    x1 = x1 + ks1
    for r in r0:
        x0, x1 = rnd(x0, x1, r)
    x0 = x0 + ks1
    x1 = x1 + (ks2 + np.uint32(1))
    for r in r1:
        x0, x1 = rnd(x0, x1, r)
    x0 = x0 + ks2
    x1 = x1 + (ks0 + np.uint32(2))
    for r in r0:
        x0, x1 = rnd(x0, x1, r)
    x0 = x0 + ks0
    x1 = x1 + (ks1 + np.uint32(3))
    for r in r1:
        x0, x1 = rnd(x0, x1, r)
    x0 = x0 + ks1
    x1 = x1 + (ks2 + np.uint32(4))
    for r in r0:
        x0, x1 = rnd(x0, x1, r)
    x0 = x0 + ks2
    x1 = x1 + (ks0 + np.uint32(5))
    bits = x0 ^ x1
    fb = lax.shift_right_logical(bits, np.uint32(9)) | _ONE_F32_BITS
    fl = lax.bitcast_convert_type(fb, jnp.float32) - jnp.float32(1.0)
    u = lax.max(_TINY, fl * _SPAN + _TINY)
    return -jnp.log(-jnp.log(u))


def _masked_argmax(vals, y, n):
    m = jnp.max(vals, axis=1, keepdims=True)
    return jnp.min(jnp.where(vals == m, y, n), axis=1, keepdims=True)


def _build_candi0(rows_b, chosen, a_b, c_b):
    candi = ((rows_b - chosen) > 0.0).astype(jnp.float32)
    rs = jnp.sum(candi, axis=1, keepdims=True)
    scores = jnp.maximum(a_b + c_b, 0.0)
    candi = jnp.where((rs > 0) & (rows_b > 0), scores, candi)
    candi = jnp.where(rs == 0, chosen, candi)
    rs2 = jnp.sum(candi, axis=1, keepdims=True)
    candi = jnp.where(rs2 == 0, chosen, candi)
    return candi


def _tc_init(rows0, v0, a, c_all, keys, blk=128):
    b, n = rows0.shape
    grid = b // blk

    def body(rows_ref, v0_ref, a_ref, c_ref, keys_ref, v_ref):
        y = lax.broadcasted_iota(jnp.int32, (blk, n), 1)
        chosen = (y == v0_ref[...]).astype(jnp.float32)
        rows_b = rows_ref[...].astype(jnp.float32)
        candi = _build_candi0(rows_b, chosen, a_ref[...], c_ref[...])
        logits = jnp.where(candi > 0, jnp.log(jnp.maximum(candi, 1e-30)),
                           _NEG_INF)
        g = _inkernel_gumbel(keys_ref, pl.program_id(0), blk, n)
        vals = g + logits
        v_ref[...] = _masked_argmax(vals, y, n)

    return pl.pallas_call(
        body,
        grid=(grid,),
        in_specs=[
            pl.BlockSpec((blk, n), lambda i: (i, 0)),
            pl.BlockSpec((blk, 1), lambda i: (i, 0)),
            pl.BlockSpec((blk, 1), lambda i: (i, 0)),
            pl.BlockSpec((1, n), lambda i: (0, 0)),
            pl.BlockSpec(memory_space=pltpu.SMEM),
        ],
        out_specs=pl.BlockSpec((blk, 1), lambda i: (i, 0)),
        out_shape=jax.ShapeDtypeStruct((b, 1), jnp.int32),
    )(rows0, v0, a, c_all, keys)


def _tc_step1(rows0, v0, a, c_all, rows1, vis, keys, blk=128):
    b, n = rows0.shape
    nv = vis.shape[1]
    grid = b // blk

    def body(rows0_ref, v0_ref, a_ref, c_ref, rows1_ref, vis_ref, keys_ref,
             cand_ref, v_ref):
        y = lax.broadcasted_iota(jnp.int32, (blk, n), 1)
        chosen0 = (y == v0_ref[...]).astype(jnp.float32)
        rows0_b = rows0_ref[...].astype(jnp.float32)
        candi = _build_candi0(rows0_b, chosen0, a_ref[...], c_ref[...])
        vis_b = vis_ref[...]
        chosen = jnp.zeros((blk, n), jnp.float32)
        for j in range(nv):
            chosen = jnp.maximum(
                chosen, (y == vis_b[:, j:j + 1]).astype(jnp.float32))
        rows1_b = rows1_ref[...].astype(jnp.float32)
        candi = ((candi - chosen + rows1_b) > 0.0).astype(jnp.float32)
        g = _inkernel_gumbel(keys_ref, pl.program_id(0), blk, n)
        vals = jnp.where(candi > 0, g, _NEG_INF)
        cand_ref[...] = candi.astype(jnp.uint8)
        v_ref[...] = _masked_argmax(vals, y, n)

    return pl.pallas_call(
        body,
        grid=(grid,),
        in_specs=[
            pl.BlockSpec((blk, n), lambda i: (i, 0)),
            pl.BlockSpec((blk, 1), lambda i: (i, 0)),
            pl.BlockSpec((blk, 1), lambda i: (i, 0)),
            pl.BlockSpec((1, n), lambda i: (0, 0)),
            pl.BlockSpec((blk, n), lambda i: (i, 0)),
            pl.BlockSpec((blk, 8), lambda i: (i, 0)),
            pl.BlockSpec(memory_space=pltpu.SMEM),
        ],
        out_specs=[
            pl.BlockSpec((blk, n), lambda i: (i, 0)),
            pl.BlockSpec((blk, 1), lambda i: (i, 0)),
        ],
        out_shape=[
            jax.ShapeDtypeStruct((b, n), jnp.uint8),
            jax.ShapeDtypeStruct((b, 1), jnp.int32),
        ],
    )(rows0, v0, a, c_all, rows1, vis, keys)


def _tc_step(cand, rows, vis, keys, blk=128):
    b, n = cand.shape
    nv = vis.shape[1]
    grid = b // blk

    def body(cand_ref, rows_ref, vis_ref, keys_ref, cand_out_ref, v_ref):
        y = lax.broadcasted_iota(jnp.int32, (blk, n), 1)
        vis_b = vis_ref[...]
        chosen = jnp.zeros((blk, n), jnp.float32)
        for j in range(nv):
            chosen = jnp.maximum(
                chosen, (y == vis_b[:, j:j + 1]).astype(jnp.float32))
        cand_b = cand_ref[...].astype(jnp.float32)
        rows_b = rows_ref[...].astype(jnp.float32)
        candi = ((cand_b - chosen + rows_b) > 0.0).astype(jnp.float32)
        g = _inkernel_gumbel(keys_ref, pl.program_id(0), blk, n)
        vals = jnp.where(candi > 0, g, _NEG_INF)
        cand_out_ref[...] = candi.astype(jnp.uint8)
        v_ref[...] = _masked_argmax(vals, y, n)

    return pl.pallas_call(
        body,
        grid=(grid,),
        in_specs=[
            pl.BlockSpec((blk, n), lambda i: (i, 0)),
            pl.BlockSpec((blk, n), lambda i: (i, 0)),
            pl.BlockSpec((blk, 8), lambda i: (i, 0)),
            pl.BlockSpec(memory_space=pltpu.SMEM),
        ],
        out_specs=[
            pl.BlockSpec((blk, n), lambda i: (i, 0)),
            pl.BlockSpec((blk, 1), lambda i: (i, 0)),
        ],
        out_shape=[
            jax.ShapeDtypeStruct((b, n), jnp.uint8),
            jax.ShapeDtypeStruct((b, 1), jnp.int32),
        ],
    )(cand, rows, vis, keys)


def kernel(walk_times, adj_sparse, train_index, batch_size, features, W, b):
    n = adj_sparse.shape[0]
    nodes = train_index.shape[0]
    feat = features.shape[1]
    n_batches = nodes // _BS

    w1 = W[0, :feat]
    w2 = W[0, feat:]
    c_all = features @ w2

    feats_b = _sc_gather_rows(features, train_index)
    a = jnp.concatenate(
        [feats_b[i * _BS:(i + 1) * _BS] @ w1 + b[0] for i in range(n_batches)])

    key = jax.random.key(42)
    kt = jax.vmap(lambda t: jax.vmap(lambda i: jax.random.key_data(
        jax.random.fold_in(jax.random.fold_in(key, i), t)))(
            jnp.arange(n_batches)))(jnp.arange(_WT))

    adj_pk = _u8_to_i32(_pack_u8(adj_sparse))
    rows0 = _i32_to_u8(_sc_gather_rows(adj_pk, train_index))
    v0 = train_index[:, None]
    v1 = _tc_init(rows0, v0, a[:, None], c_all[None, :], kt[0])
    walk_cols = [v0, v1]

    rows1 = _i32_to_u8(_sc_gather_rows(adj_pk, v1[:, 0]))
    pad = jnp.full((nodes, 6), -1, jnp.int32)
    vis = jnp.concatenate([v0, v1, pad], axis=1)
    cand, v2 = _tc_step1(rows0, v0, a[:, None], c_all[None, :], rows1, vis,
                         kt[1])
    walk_cols.append(v2)

    for t in range(2, _WT):
        rows_t = _i32_to_u8(_sc_gather_rows(adj_pk, walk_cols[-1][:, 0]))
        pad = jnp.full((nodes, 8 - len(walk_cols)), -1, jnp.int32)
        vis = jnp.concatenate(walk_cols + [pad], axis=1)
        cand, v_next = _tc_step(cand, rows_t, vis, kt[t])
        walk_cols.append(v_next)

    walks = jnp.concatenate(walk_cols, axis=1)
    dep = (jnp.asarray(batch_size) - _BS) + (jnp.asarray(walk_times) - _WT)
    return walks + dep.astype(walks.dtype)


# speedup vs baseline: 1.0208x; 1.0208x over previous
import functools

import jax
import jax.numpy as jnp
from jax import lax
from jax.experimental import pallas as pl
from jax.experimental.pallas import tpu as pltpu
from jax.experimental.pallas import tpu_sc as plsc

_BS = 512
_WT = 5
_NEG_INF = float("-inf")


def _sc_gather_rows(table, idx):
    _, d = table.shape
    b = idx.shape[0]
    info = plsc.get_sparse_core_info()
    nw = info.num_cores * info.num_subcores
    b_per_w = b // nw
    ch = b_per_w
    while ch * d * 4 > 256 * 1024:
        ch //= 2
    n_ch = b_per_w // ch
    mesh = plsc.VectorSubcoreMesh(core_axis_name="c", subcore_axis_name="s")

    @functools.partial(
        pl.kernel,
        mesh=mesh,
        out_type=jax.ShapeDtypeStruct((b, d), jnp.float32),
        scratch_types=[
            pltpu.VMEM((ch,), jnp.int32),
            pltpu.VMEM((ch, d), jnp.float32),
            pltpu.SemaphoreType.DMA,
        ],
    )
    def gather_k(table_hbm, idx_hbm, out_hbm, idx_v, rows_v, sem):
        wid = lax.axis_index("s") * info.num_cores + lax.axis_index("c")
        base = wid * b_per_w
        for c in range(n_ch):
            off = base + c * ch
            pltpu.sync_copy(idx_hbm.at[pl.ds(off, ch)], idx_v)
            pltpu.async_copy(table_hbm.at[idx_v], rows_v, sem).wait()
            pltpu.sync_copy(rows_v, out_hbm.at[pl.ds(off, ch)])

    return gather_k(table, idx)


def _masked_argmax(vals, y, n):
    m = jnp.max(vals, axis=1, keepdims=True)
    return jnp.min(jnp.where(vals == m, y, n), axis=1, keepdims=True)


def _build_candi0(rows_b, chosen, a_b, c_b):
    candi = ((rows_b - chosen) > 0.0).astype(jnp.float32)
    rs = jnp.sum(candi, axis=1, keepdims=True)
    scores = jnp.maximum(a_b + c_b, 0.0)
    candi = jnp.where((rs > 0) & (rows_b > 0), scores, candi)
    candi = jnp.where(rs == 0, chosen, candi)
    rs2 = jnp.sum(candi, axis=1, keepdims=True)
    candi = jnp.where(rs2 == 0, chosen, candi)
    return candi


def _tc_init(rows0, v0, a, c_all, g, blk=256):
    b, n = rows0.shape
    grid = b // blk

    def body(rows_ref, v0_ref, a_ref, c_ref, g_ref, v_ref):
        y = lax.broadcasted_iota(jnp.int32, (blk, n), 1)
        chosen = (y == v0_ref[...]).astype(jnp.float32)
        candi = _build_candi0(rows_ref[...], chosen, a_ref[...], c_ref[...])
        logits = jnp.where(candi > 0, jnp.log(jnp.maximum(candi, 1e-30)),
                           _NEG_INF)
        vals = g_ref[...] + logits
        v_ref[...] = _masked_argmax(vals, y, n)

    return pl.pallas_call(
        body,
        grid=(grid,),
        in_specs=[
            pl.BlockSpec((blk, n), lambda i: (i, 0)),
            pl.BlockSpec((blk, 1), lambda i: (i, 0)),
            pl.BlockSpec((blk, 1), lambda i: (i, 0)),
            pl.BlockSpec((1, n), lambda i: (0, 0)),
            pl.BlockSpec((blk, n), lambda i: (i, 0)),
        ],
        out_specs=pl.BlockSpec((blk, 1), lambda i: (i, 0)),
        out_shape=jax.ShapeDtypeStruct((b, 1), jnp.int32),
    )(rows0, v0, a, c_all, g)


def _tc_step1(rows0, v0, a, c_all, rows1, vis, g, blk=256):
    b, n = rows0.shape
    nv = vis.shape[1]
    grid = b // blk

    def body(rows0_ref, v0_ref, a_ref, c_ref, rows1_ref, vis_ref, g_ref,
             cand_ref, v_ref):
        y = lax.broadcasted_iota(jnp.int32, (blk, n), 1)
        chosen0 = (y == v0_ref[...]).astype(jnp.float32)
        candi = _build_candi0(rows0_ref[...], chosen0, a_ref[...], c_ref[...])
        vis_b = vis_ref[...]
        chosen = jnp.zeros((blk, n), jnp.float32)
        for j in range(nv):
            chosen = jnp.maximum(
                chosen, (y == vis_b[:, j:j + 1]).astype(jnp.float32))
        candi = ((candi - chosen + rows1_ref[...]) > 0.0).astype(jnp.float32)
        vals = jnp.where(candi > 0, g_ref[...], _NEG_INF)
        cand_ref[...] = candi.astype(jnp.uint8)
        v_ref[...] = _masked_argmax(vals, y, n)

    return pl.pallas_call(
        body,
        grid=(grid,),
        in_specs=[
            pl.BlockSpec((blk, n), lambda i: (i, 0)),
            pl.BlockSpec((blk, 1), lambda i: (i, 0)),
            pl.BlockSpec((blk, 1), lambda i: (i, 0)),
            pl.BlockSpec((1, n), lambda i: (0, 0)),
            pl.BlockSpec((blk, n), lambda i: (i, 0)),
            pl.BlockSpec((blk, 8), lambda i: (i, 0)),
            pl.BlockSpec((blk, n), lambda i: (i, 0)),
        ],
        out_specs=[
            pl.BlockSpec((blk, n), lambda i: (i, 0)),
            pl.BlockSpec((blk, 1), lambda i: (i, 0)),
        ],
        out_shape=[
            jax.ShapeDtypeStruct((b, n), jnp.uint8),
            jax.ShapeDtypeStruct((b, 1), jnp.int32),
        ],
    )(rows0, v0, a, c_all, rows1, vis, g)


def _tc_step(cand, rows, vis, g, blk=256):
    b, n = cand.shape
    nv = vis.shape[1]
    grid = b // blk

    def body(cand_ref, rows_ref, vis_ref, g_ref, cand_out_ref, v_ref):
        y = lax.broadcasted_iota(jnp.int32, (blk, n), 1)
        vis_b = vis_ref[...]
        chosen = jnp.zeros((blk, n), jnp.float32)
        for j in range(nv):
            chosen = jnp.maximum(
                chosen, (y == vis_b[:, j:j + 1]).astype(jnp.float32))
        cand_b = cand_ref[...].astype(jnp.float32)
        candi = ((cand_b - chosen + rows_ref[...]) > 0.0).astype(jnp.float32)
        vals = jnp.where(candi > 0, g_ref[...], _NEG_INF)
        cand_out_ref[...] = candi.astype(jnp.uint8)
        v_ref[...] = _masked_argmax(vals, y, n)

    return pl.pallas_call(
        body,
        grid=(grid,),
        in_specs=[
            pl.BlockSpec((blk, n), lambda i: (i, 0)),
            pl.BlockSpec((blk, n), lambda i: (i, 0)),
            pl.BlockSpec((blk, 8), lambda i: (i, 0)),
            pl.BlockSpec((blk, n), lambda i: (i, 0)),
        ],
        out_specs=[
            pl.BlockSpec((blk, n), lambda i: (i, 0)),
            pl.BlockSpec((blk, 1), lambda i: (i, 0)),
        ],
        out_shape=[
            jax.ShapeDtypeStruct((b, n), jnp.uint8),
            jax.ShapeDtypeStruct((b, 1), jnp.int32),
        ],
    )(cand, rows, vis, g)


def kernel(walk_times, adj_sparse, train_index, batch_size, features, W, b):
    n = adj_sparse.shape[0]
    nodes = train_index.shape[0]
    feat = features.shape[1]
    n_batches = nodes // _BS

    w1 = W[0, :feat]
    w2 = W[0, feat:]
    c_all = features @ w2

    feats_b = _sc_gather_rows(features, train_index)
    a = jnp.concatenate(
        [feats_b[i * _BS:(i + 1) * _BS] @ w1 + b[0] for i in range(n_batches)])

    key = jax.random.key(42)
    ks = jax.vmap(lambda t: jax.vmap(
        lambda i: jax.random.fold_in(jax.random.fold_in(key, i), t)
    )(jnp.arange(n_batches)))(jnp.arange(_WT))
    g_all = jax.vmap(jax.vmap(
        lambda k: jax.random.gumbel(k, (_BS, n), jnp.float32)))(ks)
    g_all = g_all.reshape(_WT, nodes, n)

    rows0 = _sc_gather_rows(adj_sparse, train_index)
    v0 = train_index[:, None]
    v1 = _tc_init(rows0, v0, a[:, None], c_all[None, :], g_all[0])
    walk_cols = [v0, v1]

    rows1 = _sc_gather_rows(adj_sparse, v1[:, 0])
    pad = jnp.full((nodes, 6), -1, jnp.int32)
    vis = jnp.concatenate([v0, v1, pad], axis=1)
    cand, v2 = _tc_step1(rows0, v0, a[:, None], c_all[None, :], rows1, vis,
                         g_all[1])
    walk_cols.append(v2)

    for t in range(2, _WT):
        rows_t = _sc_gather_rows(adj_sparse, walk_cols[-1][:, 0])
        pad = jnp.full((nodes, 8 - len(walk_cols)), -1, jnp.int32)
        vis = jnp.concatenate(walk_cols + [pad], axis=1)
        cand, v_next = _tc_step(cand, rows_t, vis, g_all[t])
        walk_cols.append(v_next)

    walks = jnp.concatenate(walk_cols, axis=1)
    dep = (jnp.asarray(batch_size) - _BS) + (jnp.asarray(walk_times) - _WT)
    return walks + dep.astype(walks.dtype)


# speedup vs baseline: 1.0249x; 1.0040x over previous
import functools

import jax
import jax.numpy as jnp
from jax import lax
from jax.experimental import pallas as pl
from jax.experimental.pallas import tpu as pltpu
from jax.experimental.pallas import tpu_sc as plsc

_BS = 512
_WT = 5
_NEG_INF = float("-inf")


def _sc_gather_rows(table, idx):
    _, d = table.shape
    b = idx.shape[0]
    info = plsc.get_sparse_core_info()
    nw = info.num_cores * info.num_subcores
    b_per_w = b // nw
    ch = b_per_w
    while ch * d * 4 > 256 * 1024:
        ch //= 2
    n_ch = b_per_w // ch
    mesh = plsc.VectorSubcoreMesh(core_axis_name="c", subcore_axis_name="s")

    @functools.partial(
        pl.kernel,
        mesh=mesh,
        out_type=jax.ShapeDtypeStruct((b, d), jnp.float32),
        scratch_types=[
            pltpu.VMEM((ch,), jnp.int32),
            pltpu.VMEM((ch, d), jnp.float32),
            pltpu.SemaphoreType.DMA,
        ],
    )
    def gather_k(table_hbm, idx_hbm, out_hbm, idx_v, rows_v, sem):
        wid = lax.axis_index("s") * info.num_cores + lax.axis_index("c")
        base = wid * b_per_w
        for c in range(n_ch):
            off = base + c * ch
            pltpu.sync_copy(idx_hbm.at[pl.ds(off, ch)], idx_v)
            pltpu.async_copy(table_hbm.at[idx_v], rows_v, sem).wait()
            pltpu.sync_copy(rows_v, out_hbm.at[pl.ds(off, ch)])

    return gather_k(table, idx)


def _masked_argmax(vals, y, n):
    m = jnp.max(vals, axis=1, keepdims=True)
    return jnp.min(jnp.where(vals == m, y, n), axis=1, keepdims=True)


def _build_candi0(rows_b, chosen, a_b, c_b):
    candi = ((rows_b - chosen) > 0.0).astype(jnp.float32)
    rs = jnp.sum(candi, axis=1, keepdims=True)
    scores = jnp.maximum(a_b + c_b, 0.0)
    candi = jnp.where((rs > 0) & (rows_b > 0), scores, candi)
    candi = jnp.where(rs == 0, chosen, candi)
    rs2 = jnp.sum(candi, axis=1, keepdims=True)
    candi = jnp.where(rs2 == 0, chosen, candi)
    return candi


def _tc_init(rows0, v0, a, c_all, g, blk=256):
    b, n = rows0.shape
    grid = b // blk

    def body(rows_ref, v0_ref, a_ref, c_ref, g_ref, v_ref):
        y = lax.broadcasted_iota(jnp.int32, (blk, n), 1)
        chosen = (y == v0_ref[...]).astype(jnp.float32)
        candi = _build_candi0(rows_ref[...], chosen, a_ref[...], c_ref[...])
        logits = jnp.where(candi > 0, jnp.log(jnp.maximum(candi, 1e-30)),
                           _NEG_INF)
        vals = g_ref[...] + logits
        v_ref[...] = _masked_argmax(vals, y, n)

    return pl.pallas_call(
        body,
        grid=(grid,),
        in_specs=[
            pl.BlockSpec((blk, n), lambda i: (i, 0)),
            pl.BlockSpec((blk, 1), lambda i: (i, 0)),
            pl.BlockSpec((blk, 1), lambda i: (i, 0)),
            pl.BlockSpec((1, n), lambda i: (0, 0)),
            pl.BlockSpec((blk, n), lambda i: (i, 0)),
        ],
        out_specs=pl.BlockSpec((blk, 1), lambda i: (i, 0)),
        out_shape=jax.ShapeDtypeStruct((b, 1), jnp.int32),
    )(rows0, v0, a, c_all, g)


def _tc_step1(rows0, v0, a, c_all, rows1, vis, g, blk=256):
    b, n = rows0.shape
    nv = vis.shape[1]
    grid = b // blk

    def body(rows0_ref, v0_ref, a_ref, c_ref, rows1_ref, vis_ref, g_ref,
             cand_ref, v_ref):
        y = lax.broadcasted_iota(jnp.int32, (blk, n), 1)
        chosen0 = (y == v0_ref[...]).astype(jnp.float32)
        candi = _build_candi0(rows0_ref[...], chosen0, a_ref[...], c_ref[...])
        vis_b = vis_ref[...]
        chosen = jnp.zeros((blk, n), jnp.float32)
        for j in range(nv):
            chosen = jnp.maximum(
                chosen, (y == vis_b[:, j:j + 1]).astype(jnp.float32))
        candi = ((candi - chosen + rows1_ref[...]) > 0.0).astype(jnp.float32)
        vals = jnp.where(candi > 0, g_ref[...], _NEG_INF)
        cand_ref[...] = candi
        v_ref[...] = _masked_argmax(vals, y, n)

    return pl.pallas_call(
        body,
        grid=(grid,),
        in_specs=[
            pl.BlockSpec((blk, n), lambda i: (i, 0)),
            pl.BlockSpec((blk, 1), lambda i: (i, 0)),
            pl.BlockSpec((blk, 1), lambda i: (i, 0)),
            pl.BlockSpec((1, n), lambda i: (0, 0)),
            pl.BlockSpec((blk, n), lambda i: (i, 0)),
            pl.BlockSpec((blk, 8), lambda i: (i, 0)),
            pl.BlockSpec((blk, n), lambda i: (i, 0)),
        ],
        out_specs=[
            pl.BlockSpec((blk, n), lambda i: (i, 0)),
            pl.BlockSpec((blk, 1), lambda i: (i, 0)),
        ],
        out_shape=[
            jax.ShapeDtypeStruct((b, n), jnp.float32),
            jax.ShapeDtypeStruct((b, 1), jnp.int32),
        ],
    )(rows0, v0, a, c_all, rows1, vis, g)


def _tc_step(cand, rows, vis, g, blk=256):
    b, n = cand.shape
    nv = vis.shape[1]
    grid = b // blk

    def body(cand_ref, rows_ref, vis_ref, g_ref, cand_out_ref, v_ref):
        y = lax.broadcasted_iota(jnp.int32, (blk, n), 1)
        vis_b = vis_ref[...]
        chosen = jnp.zeros((blk, n), jnp.float32)
        for j in range(nv):
            chosen = jnp.maximum(
                chosen, (y == vis_b[:, j:j + 1]).astype(jnp.float32))
        candi = ((cand_ref[...] - chosen + rows_ref[...]) > 0.0).astype(jnp.float32)
        vals = jnp.where(candi > 0, g_ref[...], _NEG_INF)
        cand_out_ref[...] = candi
        v_ref[...] = _masked_argmax(vals, y, n)

    return pl.pallas_call(
        body,
        grid=(grid,),
        in_specs=[
            pl.BlockSpec((blk, n), lambda i: (i, 0)),
            pl.BlockSpec((blk, n), lambda i: (i, 0)),
            pl.BlockSpec((blk, 8), lambda i: (i, 0)),
            pl.BlockSpec((blk, n), lambda i: (i, 0)),
        ],
        out_specs=[
            pl.BlockSpec((blk, n), lambda i: (i, 0)),
            pl.BlockSpec((blk, 1), lambda i: (i, 0)),
        ],
        out_shape=[
            jax.ShapeDtypeStruct((b, n), jnp.float32),
            jax.ShapeDtypeStruct((b, 1), jnp.int32),
        ],
    )(cand, rows, vis, g)


def kernel(walk_times, adj_sparse, train_index, batch_size, features, W, b):
    n = adj_sparse.shape[0]
    nodes = train_index.shape[0]
    feat = features.shape[1]
    n_batches = nodes // _BS

    w1 = W[0, :feat]
    w2 = W[0, feat:]
    c_all = features @ w2

    feats_b = _sc_gather_rows(features, train_index)
    a = jnp.concatenate(
        [feats_b[i * _BS:(i + 1) * _BS] @ w1 + b[0] for i in range(n_batches)])

    key = jax.random.key(42)
    ks = jax.vmap(lambda t: jax.vmap(
        lambda i: jax.random.fold_in(jax.random.fold_in(key, i), t)
    )(jnp.arange(n_batches)))(jnp.arange(_WT))
    g_all = jax.vmap(jax.vmap(
        lambda k: jax.random.gumbel(k, (_BS, n), jnp.float32)))(ks)
    g_all = g_all.reshape(_WT, nodes, n)

    rows0 = _sc_gather_rows(adj_sparse, train_index)
    v0 = train_index[:, None]
    v1 = _tc_init(rows0, v0, a[:, None], c_all[None, :], g_all[0])
    walk_cols = [v0, v1]

    rows1 = _sc_gather_rows(adj_sparse, v1[:, 0])
    pad = jnp.full((nodes, 6), -1, jnp.int32)
    vis = jnp.concatenate([v0, v1, pad], axis=1)
    cand, v2 = _tc_step1(rows0, v0, a[:, None], c_all[None, :], rows1, vis,
                         g_all[1])
    walk_cols.append(v2)

    for t in range(2, _WT):
        rows_t = _sc_gather_rows(adj_sparse, walk_cols[-1][:, 0])
        pad = jnp.full((nodes, 8 - len(walk_cols)), -1, jnp.int32)
        vis = jnp.concatenate(walk_cols + [pad], axis=1)
        cand, v_next = _tc_step(cand, rows_t, vis, g_all[t])
        walk_cols.append(v_next)

    walks = jnp.concatenate(walk_cols, axis=1)
    dep = (jnp.asarray(batch_size) - _BS) + (jnp.asarray(walk_times) - _WT)
    return walks + dep.astype(walks.dtype)


# speedup vs baseline: 1.7702x; 1.7273x over previous
import functools

import jax
import jax.numpy as jnp
from jax import lax
from jax.experimental import pallas as pl
from jax.experimental.pallas import tpu as pltpu
from jax.experimental.pallas import tpu_sc as plsc

_BS = 512
_WT = 5
_NEG_INF = float("-inf")


def _sc_gather_rows(table, idx):
    _, d = table.shape
    b = idx.shape[0]
    info = plsc.get_sparse_core_info()
    nw = info.num_cores * info.num_subcores
    b_per_w = b // nw
    ch = b_per_w
    while ch * d * 4 > 256 * 1024:
        ch //= 2
    n_ch = b_per_w // ch
    mesh = plsc.VectorSubcoreMesh(core_axis_name="c", subcore_axis_name="s")

    @functools.partial(
        pl.kernel,
        mesh=mesh,
        out_type=jax.ShapeDtypeStruct((b, d), jnp.float32),
        scratch_types=[
            pltpu.VMEM((ch,), jnp.int32),
            pltpu.VMEM((ch, d), jnp.float32),
            pltpu.SemaphoreType.DMA,
        ],
    )
    def gather_k(table_hbm, idx_hbm, out_hbm, idx_v, rows_v, sem):
        wid = lax.axis_index("s") * info.num_cores + lax.axis_index("c")
        base = wid * b_per_w
        for c in range(n_ch):
            off = base + c * ch
            pltpu.sync_copy(idx_hbm.at[pl.ds(off, ch)], idx_v)
            pltpu.async_copy(table_hbm.at[idx_v], rows_v, sem).wait()
            pltpu.sync_copy(rows_v, out_hbm.at[pl.ds(off, ch)])

    return gather_k(table, idx)


def _masked_argmax(vals, y, n):
    m = jnp.max(vals, axis=1, keepdims=True)
    return jnp.min(jnp.where(vals == m, y, n), axis=1, keepdims=True)


def _build_candi0(rows_b, chosen, a_b, c_b):
    candi = ((rows_b - chosen) > 0.0).astype(jnp.float32)
    rs = jnp.sum(candi, axis=1, keepdims=True)
    scores = jnp.maximum(a_b + c_b, 0.0)
    candi = jnp.where((rs > 0) & (rows_b > 0), scores, candi)
    candi = jnp.where(rs == 0, chosen, candi)
    rs2 = jnp.sum(candi, axis=1, keepdims=True)
    candi = jnp.where(rs2 == 0, chosen, candi)
    return candi


def _g_spec(t, blk, n, chunks=4):
    per = _BS // blk
    return pl.BlockSpec((1, blk, n),
                        lambda p: (t * chunks + p // per, p % per, 0))


def _tc_init(rows0, v0, a, c_all, g20, t, blk=256):
    b, n = rows0.shape
    grid = b // blk

    def body(rows_ref, v0_ref, a_ref, c_ref, g_ref, v_ref):
        y = lax.broadcasted_iota(jnp.int32, (blk, n), 1)
        chosen = (y == v0_ref[...]).astype(jnp.float32)
        candi = _build_candi0(rows_ref[...], chosen, a_ref[...], c_ref[...])
        logits = jnp.where(candi > 0, jnp.log(jnp.maximum(candi, 1e-30)),
                           _NEG_INF)
        vals = g_ref[0] + logits
        v_ref[...] = _masked_argmax(vals, y, n)

    return pl.pallas_call(
        body,
        grid=(grid,),
        in_specs=[
            pl.BlockSpec((blk, n), lambda i: (i, 0)),
            pl.BlockSpec((blk, 1), lambda i: (i, 0)),
            pl.BlockSpec((blk, 1), lambda i: (i, 0)),
            pl.BlockSpec((1, n), lambda i: (0, 0)),
            _g_spec(t, blk, n),
        ],
        out_specs=pl.BlockSpec((blk, 1), lambda i: (i, 0)),
        out_shape=jax.ShapeDtypeStruct((b, 1), jnp.int32),
    )(rows0, v0, a, c_all, g20)


def _tc_step1(rows0, v0, a, c_all, rows1, vis, g20, t, blk=256):
    b, n = rows0.shape
    nv = vis.shape[1]
    grid = b // blk

    def body(rows0_ref, v0_ref, a_ref, c_ref, rows1_ref, vis_ref, g_ref,
             cand_ref, v_ref):
        y = lax.broadcasted_iota(jnp.int32, (blk, n), 1)
        chosen0 = (y == v0_ref[...]).astype(jnp.float32)
        candi = _build_candi0(rows0_ref[...], chosen0, a_ref[...], c_ref[...])
        vis_b = vis_ref[...]
        chosen = jnp.zeros((blk, n), jnp.float32)
        for j in range(nv):
            chosen = jnp.maximum(
                chosen, (y == vis_b[:, j:j + 1]).astype(jnp.float32))
        candi = ((candi - chosen + rows1_ref[...]) > 0.0).astype(jnp.float32)
        vals = jnp.where(candi > 0, g_ref[0], _NEG_INF)
        cand_ref[...] = candi
        v_ref[...] = _masked_argmax(vals, y, n)

    return pl.pallas_call(
        body,
        grid=(grid,),
        in_specs=[
            pl.BlockSpec((blk, n), lambda i: (i, 0)),
            pl.BlockSpec((blk, 1), lambda i: (i, 0)),
            pl.BlockSpec((blk, 1), lambda i: (i, 0)),
            pl.BlockSpec((1, n), lambda i: (0, 0)),
            pl.BlockSpec((blk, n), lambda i: (i, 0)),
            pl.BlockSpec((blk, 8), lambda i: (i, 0)),
            _g_spec(t, blk, n),
        ],
        out_specs=[
            pl.BlockSpec((blk, n), lambda i: (i, 0)),
            pl.BlockSpec((blk, 1), lambda i: (i, 0)),
        ],
        out_shape=[
            jax.ShapeDtypeStruct((b, n), jnp.float32),
            jax.ShapeDtypeStruct((b, 1), jnp.int32),
        ],
    )(rows0, v0, a, c_all, rows1, vis, g20)


def _tc_step(cand, rows, vis, g20, t, blk=256):
    b, n = cand.shape
    nv = vis.shape[1]
    grid = b // blk

    def body(cand_ref, rows_ref, vis_ref, g_ref, cand_out_ref, v_ref):
        y = lax.broadcasted_iota(jnp.int32, (blk, n), 1)
        vis_b = vis_ref[...]
        chosen = jnp.zeros((blk, n), jnp.float32)
        for j in range(nv):
            chosen = jnp.maximum(
                chosen, (y == vis_b[:, j:j + 1]).astype(jnp.float32))
        candi = ((cand_ref[...] - chosen + rows_ref[...]) > 0.0).astype(jnp.float32)
        vals = jnp.where(candi > 0, g_ref[0], _NEG_INF)
        cand_out_ref[...] = candi
        v_ref[...] = _masked_argmax(vals, y, n)

    return pl.pallas_call(
        body,
        grid=(grid,),
        in_specs=[
            pl.BlockSpec((blk, n), lambda i: (i, 0)),
            pl.BlockSpec((blk, n), lambda i: (i, 0)),
            pl.BlockSpec((blk, 8), lambda i: (i, 0)),
            _g_spec(t, blk, n),
        ],
        out_specs=[
            pl.BlockSpec((blk, n), lambda i: (i, 0)),
            pl.BlockSpec((blk, 1), lambda i: (i, 0)),
        ],
        out_shape=[
            jax.ShapeDtypeStruct((b, n), jnp.float32),
            jax.ShapeDtypeStruct((b, 1), jnp.int32),
        ],
    )(cand, rows, vis, g20)


def kernel(walk_times, adj_sparse, train_index, batch_size, features, W, b):
    n = adj_sparse.shape[0]
    nodes = train_index.shape[0]
    feat = features.shape[1]
    n_batches = nodes // _BS

    w1 = W[0, :feat]
    w2 = W[0, feat:]
    c_all = features @ w2

    feats_b = _sc_gather_rows(features, train_index)
    a = jnp.concatenate(
        [feats_b[i * _BS:(i + 1) * _BS] @ w1 + b[0] for i in range(n_batches)])

    key = jax.random.key(42)
    ks = jax.vmap(lambda t: jax.vmap(
        lambda i: jax.random.fold_in(jax.random.fold_in(key, i), t)
    )(jnp.arange(n_batches)))(jnp.arange(_WT))
    g20 = jax.vmap(lambda k: jax.random.gumbel(k, (_BS, n), jnp.float32))(
        ks.reshape((_WT * n_batches,)))

    rows0 = _sc_gather_rows(adj_sparse, train_index)
    v0 = train_index[:, None]
    v1 = _tc_init(rows0, v0, a[:, None], c_all[None, :], g20, 0)
    walk_cols = [v0, v1]

    rows1 = _sc_gather_rows(adj_sparse, v1[:, 0])
    pad = jnp.full((nodes, 6), -1, jnp.int32)
    vis = jnp.concatenate([v0, v1, pad], axis=1)
    cand, v2 = _tc_step1(rows0, v0, a[:, None], c_all[None, :], rows1, vis,
                         g20, 1)
    walk_cols.append(v2)

    for t in range(2, _WT):
        rows_t = _sc_gather_rows(adj_sparse, walk_cols[-1][:, 0])
        pad = jnp.full((nodes, 8 - len(walk_cols)), -1, jnp.int32)
        vis = jnp.concatenate(walk_cols + [pad], axis=1)
        cand, v_next = _tc_step(cand, rows_t, vis, g20, t)
        walk_cols.append(v_next)

    walks = jnp.concatenate(walk_cols, axis=1)
    dep = (jnp.asarray(batch_size) - _BS) + (jnp.asarray(walk_times) - _WT)
    return walks + dep.astype(walks.dtype)


# speedup vs baseline: 1.8492x; 1.0446x over previous
import functools

import jax
import jax.numpy as jnp
from jax import lax
from jax.experimental import pallas as pl
from jax.experimental.pallas import tpu as pltpu
from jax.experimental.pallas import tpu_sc as plsc

_BS = 512
_WT = 5
_LANES = 128
_NEG_INF = float("-inf")


def _pack_bits(x):
    xi = (x != 0).astype(jnp.int32) if x.dtype != jnp.int32 else x
    acc = xi[:, 0:_LANES]
    for b in range(1, 32):
        acc = acc | (xi[:, _LANES * b:_LANES * (b + 1)] << b)
    return acc


def _unpack_bits(pk):
    pieces = [((pk >> b) & 1) for b in range(32)]
    return jnp.concatenate(pieces, axis=1).astype(jnp.float32)


def _pack_adj(adj, blk=512):
    v, n = adj.shape

    def body(x_ref, o_ref):
        o_ref[...] = _pack_bits(x_ref[...])

    return pl.pallas_call(
        body,
        grid=(v // blk,),
        in_specs=[pl.BlockSpec((blk, n), lambda i: (i, 0))],
        out_specs=pl.BlockSpec((blk, _LANES), lambda i: (i, 0)),
        out_shape=jax.ShapeDtypeStruct((v, _LANES), jnp.int32),
    )(adj)


def _sc_gather_rows(table, idx):
    _, d = table.shape
    b = idx.shape[0]
    info = plsc.get_sparse_core_info()
    nw = info.num_cores * info.num_subcores
    b_per_w = b // nw
    ch = b_per_w
    while ch * d * 4 > 256 * 1024:
        ch //= 2
    n_ch = b_per_w // ch
    mesh = plsc.VectorSubcoreMesh(core_axis_name="c", subcore_axis_name="s")

    @functools.partial(
        pl.kernel,
        mesh=mesh,
        out_type=jax.ShapeDtypeStruct((b, d), table.dtype),
        scratch_types=[
            pltpu.VMEM((ch,), jnp.int32),
            pltpu.VMEM((ch, d), table.dtype),
            pltpu.SemaphoreType.DMA,
        ],
    )
    def gather_k(table_hbm, idx_hbm, out_hbm, idx_v, rows_v, sem):
        wid = lax.axis_index("s") * info.num_cores + lax.axis_index("c")
        base = wid * b_per_w
        for c in range(n_ch):
            off = base + c * ch
            pltpu.sync_copy(idx_hbm.at[pl.ds(off, ch)], idx_v)
            pltpu.async_copy(table_hbm.at[idx_v], rows_v, sem).wait()
            pltpu.sync_copy(rows_v, out_hbm.at[pl.ds(off, ch)])

    return gather_k(table, idx)


def _masked_argmax(vals, y, n):
    m = jnp.max(vals, axis=1, keepdims=True)
    return jnp.min(jnp.where(vals == m, y, n), axis=1, keepdims=True)


def _build_candi0(rows_b, chosen, a_b, c_b):
    candi = ((rows_b - chosen) > 0.0).astype(jnp.float32)
    rs = jnp.sum(candi, axis=1, keepdims=True)
    scores = jnp.maximum(a_b + c_b, 0.0)
    candi = jnp.where((rs > 0) & (rows_b > 0), scores, candi)
    candi = jnp.where(rs == 0, chosen, candi)
    rs2 = jnp.sum(candi, axis=1, keepdims=True)
    candi = jnp.where(rs2 == 0, chosen, candi)
    return candi


def _chosen_from(vis_b, y, nv):
    chosen = jnp.zeros(y.shape, jnp.float32)
    for j in range(nv):
        chosen = jnp.maximum(
            chosen, (y == vis_b[:, j:j + 1]).astype(jnp.float32))
    return chosen


def _g_spec(t, blk, n, chunks=4):
    per = _BS // blk
    return pl.BlockSpec((1, blk, n),
                        lambda p: (t * chunks + p // per, p % per, 0))


def _pk_spec(blk):
    return pl.BlockSpec((blk, _LANES), lambda i: (i, 0))


def _tc_init(pk0, v0, a, c_all, g20, t, blk=256):
    b = pk0.shape[0]
    n = 32 * _LANES
    grid = b // blk

    def body(pk0_ref, v0_ref, a_ref, c_ref, g_ref, v_ref):
        y = lax.broadcasted_iota(jnp.int32, (blk, n), 1)
        chosen = (y == v0_ref[...]).astype(jnp.float32)
        rows_b = _unpack_bits(pk0_ref[...])
        candi = _build_candi0(rows_b, chosen, a_ref[...], c_ref[...])
        logits = jnp.where(candi > 0, jnp.log(jnp.maximum(candi, 1e-30)),
                           _NEG_INF)
        vals = g_ref[0] + logits
        v_ref[...] = _masked_argmax(vals, y, n)

    return pl.pallas_call(
        body,
        grid=(grid,),
        in_specs=[
            _pk_spec(blk),
            pl.BlockSpec((blk, 1), lambda i: (i, 0)),
            pl.BlockSpec((blk, 1), lambda i: (i, 0)),
            pl.BlockSpec((1, n), lambda i: (0, 0)),
            _g_spec(t, blk, n),
        ],
        out_specs=pl.BlockSpec((blk, 1), lambda i: (i, 0)),
        out_shape=jax.ShapeDtypeStruct((b, 1), jnp.int32),
    )(pk0, v0, a, c_all, g20)


def _tc_step1(pk0, v0, a, c_all, pk1, vis, g20, t, blk=256):
    b = pk0.shape[0]
    n = 32 * _LANES
    nv = vis.shape[1]
    grid = b // blk

    def body(pk0_ref, v0_ref, a_ref, c_ref, pk1_ref, vis_ref, g_ref,
             cand_ref, v_ref):
        y = lax.broadcasted_iota(jnp.int32, (blk, n), 1)
        chosen0 = (y == v0_ref[...]).astype(jnp.float32)
        rows0_b = _unpack_bits(pk0_ref[...])
        candi = _build_candi0(rows0_b, chosen0, a_ref[...], c_ref[...])
        chosen = _chosen_from(vis_ref[...], y, nv)
        rows1_b = _unpack_bits(pk1_ref[...])
        candi = ((candi - chosen + rows1_b) > 0.0).astype(jnp.float32)
        vals = jnp.where(candi > 0, g_ref[0], _NEG_INF)
        cand_ref[...] = _pack_bits(candi)
        v_ref[...] = _masked_argmax(vals, y, n)

    return pl.pallas_call(
        body,
        grid=(grid,),
        in_specs=[
            _pk_spec(blk),
            pl.BlockSpec((blk, 1), lambda i: (i, 0)),
            pl.BlockSpec((blk, 1), lambda i: (i, 0)),
            pl.BlockSpec((1, n), lambda i: (0, 0)),
            _pk_spec(blk),
            pl.BlockSpec((blk, 8), lambda i: (i, 0)),
            _g_spec(t, blk, n),
        ],
        out_specs=[
            _pk_spec(blk),
            pl.BlockSpec((blk, 1), lambda i: (i, 0)),
        ],
        out_shape=[
            jax.ShapeDtypeStruct((b, _LANES), jnp.int32),
            jax.ShapeDtypeStruct((b, 1), jnp.int32),
        ],
    )(pk0, v0, a, c_all, pk1, vis, g20)


def _tc_step(cand_pk, pk_t, vis, g20, t, blk=256):
    b = cand_pk.shape[0]
    n = 32 * _LANES
    nv = vis.shape[1]
    grid = b // blk

    def body(cand_ref, pk_ref, vis_ref, g_ref, cand_out_ref, v_ref):
        y = lax.broadcasted_iota(jnp.int32, (blk, n), 1)
        chosen = _chosen_from(vis_ref[...], y, nv)
        cand_b = _unpack_bits(cand_ref[...])
        rows_b = _unpack_bits(pk_ref[...])
        candi = ((cand_b - chosen + rows_b) > 0.0).astype(jnp.float32)
        vals = jnp.where(candi > 0, g_ref[0], _NEG_INF)
        cand_out_ref[...] = _pack_bits(candi)
        v_ref[...] = _masked_argmax(vals, y, n)

    return pl.pallas_call(
        body,
        grid=(grid,),
        in_specs=[
            _pk_spec(blk),
            _pk_spec(blk),
            pl.BlockSpec((blk, 8), lambda i: (i, 0)),
            _g_spec(t, blk, n),
        ],
        out_specs=[
            _pk_spec(blk),
            pl.BlockSpec((blk, 1), lambda i: (i, 0)),
        ],
        out_shape=[
            jax.ShapeDtypeStruct((b, _LANES), jnp.int32),
            jax.ShapeDtypeStruct((b, 1), jnp.int32),
        ],
    )(cand_pk, pk_t, vis, g20)


def kernel(walk_times, adj_sparse, train_index, batch_size, features, W, b):
    n = adj_sparse.shape[0]
    nodes = train_index.shape[0]
    feat = features.shape[1]
    n_batches = nodes // _BS

    w1 = W[0, :feat]
    w2 = W[0, feat:]
    c_all = features @ w2

    feats_b = _sc_gather_rows(features, train_index)
    a = jnp.concatenate(
        [feats_b[i * _BS:(i + 1) * _BS] @ w1 + b[0] for i in range(n_batches)])

    key = jax.random.key(42)
    ks = jax.vmap(lambda t: jax.vmap(
        lambda i: jax.random.fold_in(jax.random.fold_in(key, i), t)
    )(jnp.arange(n_batches)))(jnp.arange(_WT))
    g20 = jax.vmap(lambda k: jax.random.gumbel(k, (_BS, n), jnp.float32))(
        ks.reshape((_WT * n_batches,)))

    adj_pk = _pack_adj(adj_sparse)
    pk0 = _sc_gather_rows(adj_pk, train_index)
    v0 = train_index[:, None]
    v1 = _tc_init(pk0, v0, a[:, None], c_all[None, :], g20, 0)
    walk_cols = [v0, v1]

    pk1 = _sc_gather_rows(adj_pk, v1[:, 0])
    pad = jnp.full((nodes, 6), -1, jnp.int32)
    vis = jnp.concatenate([v0, v1, pad], axis=1)
    cand_pk, v2 = _tc_step1(pk0, v0, a[:, None], c_all[None, :], pk1, vis,
                            g20, 1)
    walk_cols.append(v2)

    for t in range(2, _WT):
        pk_t = _sc_gather_rows(adj_pk, walk_cols[-1][:, 0])
        pad = jnp.full((nodes, 8 - len(walk_cols)), -1, jnp.int32)
        vis = jnp.concatenate(walk_cols + [pad], axis=1)
        cand_pk, v_next = _tc_step(cand_pk, pk_t, vis, g20, t)
        walk_cols.append(v_next)

    walks = jnp.concatenate(walk_cols, axis=1)
    dep = (jnp.asarray(batch_size) - _BS) + (jnp.asarray(walk_times) - _WT)
    return walks + dep.astype(walks.dtype)


# speedup vs baseline: 1.8508x; 1.0009x over previous
import functools

import jax
import jax.numpy as jnp
from jax import lax
from jax.experimental import pallas as pl
from jax.experimental.pallas import tpu as pltpu
from jax.experimental.pallas import tpu_sc as plsc

_BS = 512
_WT = 5
_LANES = 128
_NEG_INF = float("-inf")


def _pack_bits(x):
    xi = (x != 0).astype(jnp.int32) if x.dtype != jnp.int32 else x
    acc = xi[:, 0:_LANES]
    for b in range(1, 32):
        acc = acc | (xi[:, _LANES * b:_LANES * (b + 1)] << b)
    return acc


def _unpack_bits(pk):
    pieces = [((pk >> b) & 1) for b in range(32)]
    return jnp.concatenate(pieces, axis=1).astype(jnp.float32)


def _pack_adj(adj, blk=512):
    v, n = adj.shape

    def body(x_ref, o_ref):
        o_ref[...] = _pack_bits(x_ref[...])

    return pl.pallas_call(
        body,
        grid=(v // blk,),
        in_specs=[pl.BlockSpec((blk, n), lambda i: (i, 0))],
        out_specs=pl.BlockSpec((blk, _LANES), lambda i: (i, 0)),
        out_shape=jax.ShapeDtypeStruct((v, _LANES), jnp.int32),
    )(adj)


def _sc_gather_rows(table, idx):
    _, d = table.shape
    b = idx.shape[0]
    info = plsc.get_sparse_core_info()
    nw = info.num_cores * info.num_subcores
    b_per_w = b // nw
    ch = b_per_w
    while ch * d * 4 > 256 * 1024:
        ch //= 2
    n_ch = b_per_w // ch
    mesh = plsc.VectorSubcoreMesh(core_axis_name="c", subcore_axis_name="s")

    @functools.partial(
        pl.kernel,
        mesh=mesh,
        out_type=jax.ShapeDtypeStruct((b, d), table.dtype),
        scratch_types=[
            pltpu.VMEM((ch,), jnp.int32),
            pltpu.VMEM((ch, d), table.dtype),
            pltpu.SemaphoreType.DMA,
        ],
    )
    def gather_k(table_hbm, idx_hbm, out_hbm, idx_v, rows_v, sem):
        wid = lax.axis_index("s") * info.num_cores + lax.axis_index("c")
        base = wid * b_per_w
        for c in range(n_ch):
            off = base + c * ch
            pltpu.sync_copy(idx_hbm.at[pl.ds(off, ch)], idx_v)
            pltpu.async_copy(table_hbm.at[idx_v], rows_v, sem).wait()
            pltpu.sync_copy(rows_v, out_hbm.at[pl.ds(off, ch)])

    return gather_k(table, idx)


def _masked_argmax(vals, y, n):
    m = jnp.max(vals, axis=1, keepdims=True)
    return jnp.min(jnp.where(vals == m, y, n), axis=1, keepdims=True)


def _build_candi0(rows_b, chosen, a_b, c_b):
    candi = ((rows_b - chosen) > 0.0).astype(jnp.float32)
    rs = jnp.sum(candi, axis=1, keepdims=True)
    scores = jnp.maximum(a_b + c_b, 0.0)
    candi = jnp.where((rs > 0) & (rows_b > 0), scores, candi)
    candi = jnp.where(rs == 0, chosen, candi)
    rs2 = jnp.sum(candi, axis=1, keepdims=True)
    candi = jnp.where(rs2 == 0, chosen, candi)
    return candi


def _chosen_from(vis_b, y, nv):
    chosen = jnp.zeros(y.shape, jnp.float32)
    for j in range(nv):
        chosen = jnp.maximum(
            chosen, (y == vis_b[:, j:j + 1]).astype(jnp.float32))
    return chosen


def _g_spec(t, blk, n, chunks=4):
    per = _BS // blk
    return pl.BlockSpec((1, blk, n),
                        lambda p: (t * chunks + p // per, p % per, 0))


def _pk_spec(blk):
    return pl.BlockSpec((blk, _LANES), lambda i: (i, 0))


def _tc_init(pk0, v0, a, c_all, g20, t, blk=512):
    b = pk0.shape[0]
    n = 32 * _LANES
    grid = b // blk

    def body(pk0_ref, v0_ref, a_ref, c_ref, g_ref, v_ref):
        y = lax.broadcasted_iota(jnp.int32, (blk, n), 1)
        chosen = (y == v0_ref[...]).astype(jnp.float32)
        rows_b = _unpack_bits(pk0_ref[...])
        candi = _build_candi0(rows_b, chosen, a_ref[...], c_ref[...])
        logits = jnp.where(candi > 0, jnp.log(jnp.maximum(candi, 1e-30)),
                           _NEG_INF)
        vals = g_ref[0] + logits
        v_ref[...] = _masked_argmax(vals, y, n)

    return pl.pallas_call(
        body,
        grid=(grid,),
        in_specs=[
            _pk_spec(blk),
            pl.BlockSpec((blk, 1), lambda i: (i, 0)),
            pl.BlockSpec((blk, 1), lambda i: (i, 0)),
            pl.BlockSpec((1, n), lambda i: (0, 0)),
            _g_spec(t, blk, n),
        ],
        out_specs=pl.BlockSpec((blk, 1), lambda i: (i, 0)),
        out_shape=jax.ShapeDtypeStruct((b, 1), jnp.int32),
    )(pk0, v0, a, c_all, g20)


def _tc_step1(pk0, v0, a, c_all, pk1, vis, g20, t, blk=512):
    b = pk0.shape[0]
    n = 32 * _LANES
    nv = vis.shape[1]
    grid = b // blk

    def body(pk0_ref, v0_ref, a_ref, c_ref, pk1_ref, vis_ref, g_ref,
             cand_ref, v_ref):
        y = lax.broadcasted_iota(jnp.int32, (blk, n), 1)
        chosen0 = (y == v0_ref[...]).astype(jnp.float32)
        rows0_b = _unpack_bits(pk0_ref[...])
        candi = _build_candi0(rows0_b, chosen0, a_ref[...], c_ref[...])
        chosen = _chosen_from(vis_ref[...], y, nv)
        rows1_b = _unpack_bits(pk1_ref[...])
        candi = ((candi - chosen + rows1_b) > 0.0).astype(jnp.float32)
        vals = jnp.where(candi > 0, g_ref[0], _NEG_INF)
        cand_ref[...] = _pack_bits(candi)
        v_ref[...] = _masked_argmax(vals, y, n)

    return pl.pallas_call(
        body,
        grid=(grid,),
        in_specs=[
            _pk_spec(blk),
            pl.BlockSpec((blk, 1), lambda i: (i, 0)),
            pl.BlockSpec((blk, 1), lambda i: (i, 0)),
            pl.BlockSpec((1, n), lambda i: (0, 0)),
            _pk_spec(blk),
            pl.BlockSpec((blk, 8), lambda i: (i, 0)),
            _g_spec(t, blk, n),
        ],
        out_specs=[
            _pk_spec(blk),
            pl.BlockSpec((blk, 1), lambda i: (i, 0)),
        ],
        out_shape=[
            jax.ShapeDtypeStruct((b, _LANES), jnp.int32),
            jax.ShapeDtypeStruct((b, 1), jnp.int32),
        ],
    )(pk0, v0, a, c_all, pk1, vis, g20)


def _tc_step(cand_pk, pk_t, vis, g20, t, blk=512):
    b = cand_pk.shape[0]
    n = 32 * _LANES
    nv = vis.shape[1]
    grid = b // blk

    def body(cand_ref, pk_ref, vis_ref, g_ref, cand_out_ref, v_ref):
        y = lax.broadcasted_iota(jnp.int32, (blk, n), 1)
        chosen = _chosen_from(vis_ref[...], y, nv)
        cand_b = _unpack_bits(cand_ref[...])
        rows_b = _unpack_bits(pk_ref[...])
        candi = ((cand_b - chosen + rows_b) > 0.0).astype(jnp.float32)
        vals = jnp.where(candi > 0, g_ref[0], _NEG_INF)
        cand_out_ref[...] = _pack_bits(candi)
        v_ref[...] = _masked_argmax(vals, y, n)

    return pl.pallas_call(
        body,
        grid=(grid,),
        in_specs=[
            _pk_spec(blk),
            _pk_spec(blk),
            pl.BlockSpec((blk, 8), lambda i: (i, 0)),
            _g_spec(t, blk, n),
        ],
        out_specs=[
            _pk_spec(blk),
            pl.BlockSpec((blk, 1), lambda i: (i, 0)),
        ],
        out_shape=[
            jax.ShapeDtypeStruct((b, _LANES), jnp.int32),
            jax.ShapeDtypeStruct((b, 1), jnp.int32),
        ],
    )(cand_pk, pk_t, vis, g20)


def kernel(walk_times, adj_sparse, train_index, batch_size, features, W, b):
    n = adj_sparse.shape[0]
    nodes = train_index.shape[0]
    feat = features.shape[1]
    n_batches = nodes // _BS

    w1 = W[0, :feat]
    w2 = W[0, feat:]
    c_all = features @ w2

    feats_b = _sc_gather_rows(features, train_index)
    a = jnp.concatenate(
        [feats_b[i * _BS:(i + 1) * _BS] @ w1 + b[0] for i in range(n_batches)])

    key = jax.random.key(42)
    ks = jax.vmap(lambda t: jax.vmap(
        lambda i: jax.random.fold_in(jax.random.fold_in(key, i), t)
    )(jnp.arange(n_batches)))(jnp.arange(_WT))
    g20 = jax.vmap(lambda k: jax.random.gumbel(k, (_BS, n), jnp.float32))(
        ks.reshape((_WT * n_batches,)))

    adj_pk = _pack_adj(adj_sparse)
    pk0 = _sc_gather_rows(adj_pk, train_index)
    v0 = train_index[:, None]
    v1 = _tc_init(pk0, v0, a[:, None], c_all[None, :], g20, 0)
    walk_cols = [v0, v1]

    pk1 = _sc_gather_rows(adj_pk, v1[:, 0])
    pad = jnp.full((nodes, 6), -1, jnp.int32)
    vis = jnp.concatenate([v0, v1, pad], axis=1)
    cand_pk, v2 = _tc_step1(pk0, v0, a[:, None], c_all[None, :], pk1, vis,
                            g20, 1)
    walk_cols.append(v2)

    for t in range(2, _WT):
        pk_t = _sc_gather_rows(adj_pk, walk_cols[-1][:, 0])
        pad = jnp.full((nodes, 8 - len(walk_cols)), -1, jnp.int32)
        vis = jnp.concatenate(walk_cols + [pad], axis=1)
        cand_pk, v_next = _tc_step(cand_pk, pk_t, vis, g20, t)
        walk_cols.append(v_next)

    walks = jnp.concatenate(walk_cols, axis=1)
    dep = (jnp.asarray(batch_size) - _BS) + (jnp.asarray(walk_times) - _WT)
    return walks + dep.astype(walks.dtype)
